# baseline (speedup 1.0000x reference)
import functools

import jax
import jax.numpy as jnp
from jax import lax
from jax.experimental import pallas as pl
from jax.experimental.pallas import tpu as pltpu
from jax.experimental.pallas import tpu_sc as plsc

_ZROWS, _ZCOLS = 16384, 1024
_NW = 32
_RPW = _ZROWS // _NW
_ZCH = 32
_NCH = _RPW // _ZCH

_XGRID = 32
_XBLK = 262144 // _XGRID


def _make_z_sc():
    mesh = plsc.VectorSubcoreMesh(core_axis_name="c", subcore_axis_name="s",
                                  num_cores=2, num_subcores=16)

    @functools.partial(
        pl.kernel, mesh=mesh,
        out_type=jax.ShapeDtypeStruct((_ZROWS, _ZCOLS), jnp.float32),
        scratch_types=[
            pltpu.VMEM((_ZCH, _ZCOLS), jnp.float32),
            pltpu.VMEM((_ZCH, _ZCOLS), jnp.float32),
            pltpu.VMEM((2, 16), jnp.float32),
            pltpu.SemaphoreType.DMA((2,)),
            pltpu.SemaphoreType.DMA((2,)),
            pltpu.SemaphoreType.DMA,
        ],
    )
    def z_sc_kernel(z_hbm, w_hbm, zo_hbm, buf0, buf1, w_v, sem_i, sem_o, sem_w):
        wid = lax.axis_index("s") * 2 + lax.axis_index("c")
        base = wid * _RPW
        bufs = (buf0, buf1)

        @pl.when(wid == 0)
        def _load_w():
            pltpu.async_copy(w_hbm, w_v, sem_w).wait()

        def _patch(b):
            b[0, 0:16] = b[0, 0:16] + w_v[0, 0:16]
            b[1, 0:16] = b[1, 0:16] + w_v[1, 0:16]

        in_h = [None] * _NCH
        out_h = [None] * _NCH
        for c in range(_NCH):
            j = c % 2
            if c >= 2:
                out_h[c - 2].wait()
            in_h[c] = pltpu.async_copy(
                z_hbm.at[pl.ds(base + c * _ZCH, _ZCH)], bufs[j], sem_i.at[j])
            if c >= 1:
                p = c - 1
                in_h[p].wait()
                if p == 0:
                    @pl.when(wid == 0)
                    def _():
                        _patch(bufs[0])
                out_h[p] = pltpu.async_copy(
                    bufs[p % 2], zo_hbm.at[pl.ds(base + p * _ZCH, _ZCH)],
                    sem_o.at[p % 2])
        p = _NCH - 1
        in_h[p].wait()
        out_h[p] = pltpu.async_copy(
            bufs[p % 2], zo_hbm.at[pl.ds(base + p * _ZCH, _ZCH)],
            sem_o.at[p % 2])
        out_h[p - 1].wait()
        out_h[p].wait()

    return z_sc_kernel


_Z_SC = _make_z_sc()


def _x_kernel(x_ref, y_ref, xo_ref):
    i = pl.program_id(0)
    xo_ref[...] = x_ref[...]

    @pl.when(i == 0)
    def _fixup():
        xb = x_ref[0:16, :]
        rows = jax.lax.broadcasted_iota(jnp.int32, xb.shape, 0)
        xb = jnp.where(rows == 10, y_ref[0:1, :], xb)
        xb = jnp.where(rows == 2, y_ref[1:2, :], xb)
        xb = jnp.where(rows == 1, jnp.float32(45.0), xb)
        xo_ref[0:16, :] = xb


def kernel(x, y, z, w):
    upd = jnp.zeros((2, 16), jnp.float32)
    upd = upd.at[0, 1].set(w[2]).at[0, 2].set(w[1]).at[1, 3].set(w[0])
    z_out = _Z_SC(z, upd)
    x_out = pl.pallas_call(
        _x_kernel,
        grid=(_XGRID,),
        in_specs=[
            pl.BlockSpec((_XBLK, x.shape[1]), lambda i: (i, 0)),
            pl.BlockSpec((2, x.shape[1]), lambda i: (0, 0)),
        ],
        out_specs=pl.BlockSpec((_XBLK, x.shape[1]), lambda i: (i, 0)),
        out_shape=jax.ShapeDtypeStruct(x.shape, x.dtype),
        compiler_params=pltpu.CompilerParams(dimension_semantics=("parallel",)),
    )(x, y)
    return (x_out, z_out)


# speedup vs baseline: 1.1449x; 1.1449x over previous
import jax
import jax.numpy as jnp
from jax.experimental import pallas as pl
from jax.experimental.pallas import tpu as pltpu

_GRID = 32
_XBLK = 262144 // _GRID
_ZBLK = 16384 // _GRID
_XHEAD = 16
_ZHEAD = 8


def _fused_kernel(x_ref, y_ref, z_ref, w_ref, xo_ref, zo_ref):
    i = pl.program_id(0)

    xo_ref[...] = x_ref[...]
    zo_ref[...] = z_ref[...]

    @pl.when(i == 0)
    def _fixup():
        xb = x_ref[0:_XHEAD, :]
        rows = jax.lax.broadcasted_iota(jnp.int32, xb.shape, 0)
        xb = jnp.where(rows == 10, y_ref[0:1, :], xb)
        xb = jnp.where(rows == 2, y_ref[1:2, :], xb)
        xb = jnp.where(rows == 1, jnp.float32(45.0), xb)
        xo_ref[0:_XHEAD, :] = xb

        zb = z_ref[0:_ZHEAD, :]
        rows = jax.lax.broadcasted_iota(jnp.int32, zb.shape, 0)
        cols = jax.lax.broadcasted_iota(jnp.int32, zb.shape, 1)
        upd = jnp.where((rows == 1) & (cols == 3), w_ref[0], 0.0)
        upd = jnp.where((rows == 0) & (cols == 2), w_ref[1], upd)
        upd = jnp.where((rows == 0) & (cols == 1), w_ref[2], upd)
        zo_ref[0:_ZHEAD, :] = zb + upd


def kernel(x, y, z, w):
    return pl.pallas_call(
        _fused_kernel,
        grid=(_GRID,),
        in_specs=[
            pl.BlockSpec((_XBLK, x.shape[1]), lambda i: (i, 0)),
            pl.BlockSpec((2, x.shape[1]), lambda i: (0, 0)),
            pl.BlockSpec((_ZBLK, z.shape[1]), lambda i: (i, 0)),
            pl.BlockSpec(memory_space=pltpu.SMEM),
        ],
        out_specs=[
            pl.BlockSpec((_XBLK, x.shape[1]), lambda i: (i, 0)),
            pl.BlockSpec((_ZBLK, z.shape[1]), lambda i: (i, 0)),
        ],
        out_shape=[
            jax.ShapeDtypeStruct(x.shape, x.dtype),
            jax.ShapeDtypeStruct(z.shape, z.dtype),
        ],
        compiler_params=pltpu.CompilerParams(dimension_semantics=("parallel",)),
    )(x, y, z, w)
